# baseline (speedup 1.0000x reference)
import functools

import jax
import jax.numpy as jnp
from jax import lax
from jax.experimental import pallas as pl
from jax.experimental.pallas import tpu as pltpu
from jax.experimental.pallas import tpu_sc as plsc

B = 16384
K = 16
TW = 128
W = 8

_INFO = plsc.get_sparse_core_info()
_NC, _NS, _L = _INFO.num_cores, _INFO.num_subcores, _INFO.num_lanes
_NW = _NC * _NS
_BPW = B // _NW
_NPAIR = _BPW // _L


def _sc_body(users_hbm, items_hbm, gut_hbm, git_hbm,
             xui_out, guo, gio,
             uidx_v, iidx_v,
             ubufa_v, ibufa_v, ubufb_v, ibufb_v,
             ustage_v, istage_v, xui_v, gsema, gsemb, osem):
    wid = lax.axis_index("s") * _NC + lax.axis_index("c")
    base = wid * _BPW

    pltpu.sync_copy(users_hbm.at[pl.ds(base, _BPW)], uidx_v)
    pltpu.sync_copy(items_hbm.at[pl.ds(base, _BPW)], iidx_v)

    lanes = lax.iota(jnp.int32, _L)
    slot8 = lanes & (W - 1)
    lo_half = lanes < W
    sw8 = lanes ^ W

    def swap8(x):
        return x.at[sw8].get(mode="promise_in_bounds", unique_indices=True)

    bufs = [(ubufa_v, ibufa_v, gsema), (ubufb_v, ibufb_v, gsemb)]

    def fire(uids, iids, lane0, which):
        ubuf, ibuf, sem = bufs[which]
        for slot in range(W):
            utc = pl.multiple_of((uids[lane0 + slot] >> 7) * TW, TW)
            itc = pl.multiple_of((iids[lane0 + slot] >> 7) * TW, TW)
            pltpu.async_copy(gut_hbm.at[:, pl.ds(utc, TW)],
                             ubuf.at[slot], sem)
            pltpu.async_copy(git_hbm.at[:, pl.ds(itc, TW)],
                             ibuf.at[slot], sem)

    def wait(which):
        ubuf, ibuf, sem = bufs[which]
        for slot in range(W):
            pltpu.make_async_copy(gut_hbm.at[:, pl.ds(0, TW)],
                                  ubuf.at[slot], sem).wait()
            pltpu.make_async_copy(git_hbm.at[:, pl.ds(0, TW)],
                                  ibuf.at[slot], sem).wait()

    def compute(t, ul, il, which, second):
        ubuf, ibuf, _ = bufs[which]
        sl = pl.ds(t * _L, _L)
        acc = None
        for k in range(K):
            kf = jnp.full((_L,), k, jnp.int32)
            uvals = plsc.load_gather(ubuf, [slot8, kf, ul])
            ivals = plsc.load_gather(ibuf, [slot8, kf, il])
            if second:
                uvals = jnp.where(lo_half, ustage_v[k, sl], swap8(uvals))
                ivals = jnp.where(lo_half, istage_v[k, sl], swap8(ivals))
            ustage_v[k, sl] = uvals
            istage_v[k, sl] = ivals
            p = uvals * ivals
            acc = p if acc is None else acc + p
        if second:
            xui_v[sl] = acc
        return None

    uids0 = uidx_v[pl.ds(0, _L)]
    iids0 = iidx_v[pl.ds(0, _L)]
    fire(uids0, iids0, 0, 0)
    fire(uids0, iids0, W, 1)
    last = _NPAIR - 1

    def step(t, carry):
        sl = pl.ds(t * _L, _L)
        uids = uidx_v[sl]
        iids = iidx_v[sl]
        nsl = pl.ds(jnp.minimum(t + 1, last) * _L, _L)
        nuids = uidx_v[nsl]
        niids = iidx_v[nsl]
        ul = uids & (TW - 1)
        il = iids & (TW - 1)
        wait(0)
        compute(t, ul, il, 0, False)
        fire(nuids, niids, 0, 0)
        wait(1)
        compute(t, swap8(ul), swap8(il), 1, True)
        fire(nuids, niids, W, 1)
        return carry

    lax.fori_loop(0, _NPAIR, step, 0)
    wait(0)
    wait(1)

    obase = pl.multiple_of(base, 128)
    outs = [
        pltpu.async_copy(ustage_v, guo.at[:, pl.ds(obase, _BPW)], osem),
        pltpu.async_copy(istage_v, gio.at[:, pl.ds(obase, _BPW)], osem),
    ]
    pltpu.sync_copy(xui_v, xui_out.at[pl.ds(base, _BPW)])
    for cp in outs:
        cp.wait()


_mf_kernel = functools.partial(
    pl.kernel,
    mesh=plsc.VectorSubcoreMesh(core_axis_name="c", subcore_axis_name="s"),
    out_type=(
        jax.ShapeDtypeStruct((B,), jnp.float32),
        jax.ShapeDtypeStruct((K, B), jnp.float32),
        jax.ShapeDtypeStruct((K, B), jnp.float32),
    ),
    scratch_types=[
        pltpu.VMEM((_BPW,), jnp.int32),
        pltpu.VMEM((_BPW,), jnp.int32),
        pltpu.VMEM((W, K, TW), jnp.float32),
        pltpu.VMEM((W, K, TW), jnp.float32),
        pltpu.VMEM((W, K, TW), jnp.float32),
        pltpu.VMEM((W, K, TW), jnp.float32),
        pltpu.VMEM((K, _BPW), jnp.float32),
        pltpu.VMEM((K, _BPW), jnp.float32),
        pltpu.VMEM((_BPW,), jnp.float32),
        pltpu.SemaphoreType.DMA,
        pltpu.SemaphoreType.DMA,
        pltpu.SemaphoreType.DMA,
    ],
    compiler_params=pltpu.CompilerParams(needs_layout_passes=False),
)(_sc_body)


def kernel(users, items, Gu, Gi):
    xui, guo, gio = _mf_kernel(
        users.astype(jnp.int32), items.astype(jnp.int32), Gu.T, Gi.T)
    return (xui, guo.T, gio.T)


# speedup vs baseline: 1.0234x; 1.0234x over previous
import functools

import jax
import jax.numpy as jnp
from jax import lax
from jax.experimental import pallas as pl
from jax.experimental.pallas import tpu as pltpu
from jax.experimental.pallas import tpu_sc as plsc

B = 16384
K = 16
TW = 128

_INFO = plsc.get_sparse_core_info()
_NC, _NS, _L = _INFO.num_cores, _INFO.num_subcores, _INFO.num_lanes
_NW = _NC * _NS
_BPW = B // _NW
_NG = _BPW // _L


def _sc_body(users_hbm, items_hbm, gut_hbm, git_hbm,
             xui_out, guo, gio,
             uidx_v, iidx_v, ubuf_v, ibuf_v,
             ustage_v, istage_v, xui_v, gsem, osem):
    wid = lax.axis_index("s") * _NC + lax.axis_index("c")
    base = wid * _BPW

    pltpu.sync_copy(users_hbm.at[pl.ds(base, _BPW)], uidx_v)
    pltpu.sync_copy(items_hbm.at[pl.ds(base, _BPW)], iidx_v)

    lanes = lax.iota(jnp.int32, _L)

    def group(t, carry):
        sl = pl.ds(t * _L, _L)
        uids = uidx_v[sl]
        iids = iidx_v[sl]
        cps = []
        for slot in range(_L):
            utc = pl.multiple_of((uids[slot] >> 7) * TW, TW)
            itc = pl.multiple_of((iids[slot] >> 7) * TW, TW)
            cps.append(pltpu.async_copy(
                gut_hbm.at[:, pl.ds(utc, TW)], ubuf_v.at[slot], gsem))
            cps.append(pltpu.async_copy(
                git_hbm.at[:, pl.ds(itc, TW)], ibuf_v.at[slot], gsem))
        for cp in cps:
            cp.wait()

        ul = uids & (TW - 1)
        il = iids & (TW - 1)
        acc = None
        for k in range(K):
            kf = jnp.full((_L,), k, jnp.int32)
            uvals = plsc.load_gather(ubuf_v, [lanes, kf, ul])
            ivals = plsc.load_gather(ibuf_v, [lanes, kf, il])
            ustage_v[k, sl] = uvals
            istage_v[k, sl] = ivals
            p = uvals * ivals
            acc = p if acc is None else acc + p
        xui_v[sl] = acc
        return carry

    lax.fori_loop(0, _NG, group, 0)

    obase = pl.multiple_of(base, 128)
    outs = [
        pltpu.async_copy(ustage_v, guo.at[:, pl.ds(obase, _BPW)], osem),
        pltpu.async_copy(istage_v, gio.at[:, pl.ds(obase, _BPW)], osem),
    ]
    pltpu.sync_copy(xui_v, xui_out.at[pl.ds(base, _BPW)])
    for cp in outs:
        cp.wait()


_mf_kernel = functools.partial(
    pl.kernel,
    mesh=plsc.VectorSubcoreMesh(core_axis_name="c", subcore_axis_name="s"),
    out_type=(
        jax.ShapeDtypeStruct((B,), jnp.float32),
        jax.ShapeDtypeStruct((K, B), jnp.float32),
        jax.ShapeDtypeStruct((K, B), jnp.float32),
    ),
    scratch_types=[
        pltpu.VMEM((_BPW,), jnp.int32),
        pltpu.VMEM((_BPW,), jnp.int32),
        pltpu.VMEM((_L, K, TW), jnp.float32),
        pltpu.VMEM((_L, K, TW), jnp.float32),
        pltpu.VMEM((K, _BPW), jnp.float32),
        pltpu.VMEM((K, _BPW), jnp.float32),
        pltpu.VMEM((_BPW,), jnp.float32),
        pltpu.SemaphoreType.DMA,
        pltpu.SemaphoreType.DMA,
    ],
    compiler_params=pltpu.CompilerParams(needs_layout_passes=False),
)(_sc_body)


def kernel(users, items, Gu, Gi):
    xui, guo, gio = _mf_kernel(
        users.astype(jnp.int32), items.astype(jnp.int32), Gu.T, Gi.T)
    return (xui, guo.T, gio.T)
